# baseline (speedup 1.0000x reference)
import functools

import jax
import jax.numpy as jnp
from jax import lax
from jax.experimental import pallas as pl
from jax.experimental.pallas import tpu as pltpu
from jax.experimental.pallas import tpu_sc as plsc

N_NODES = 10000
N_EDGES = 320000
D = 128

NC, NS = 2, 16
NW = NC * NS
NPAD = 10240
ROWS_PER_TILE = NPAD // NS
K = 128
EPT = N_EDGES // NW
NCHUNK = 80
PAD_ROW = NPAD - 1

_mesh = plsc.VectorSubcoreMesh(core_axis_name="c", subcore_axis_name="s")


NHL = 8

DEG_B = 8

_DEG_KERNEL_PARAMS = dict(
    out_type=jax.ShapeDtypeStruct((NW, NHL * NPAD), jnp.float32),
    mesh=_mesh,
    compiler_params=pltpu.CompilerParams(needs_layout_passes=False),
    scratch_types=[
        pltpu.VMEM((DEG_B, K), jnp.int32),
        pltpu.VMEM((NHL * NPAD,), jnp.float32),
    ],
)


def _deg_body(dstp_hbm, zhist_hbm, degp_hbm, idx_v, hist_v):
    c = lax.axis_index("c")
    s = lax.axis_index("s")
    wid = c * NS + s
    pltpu.sync_copy(zhist_hbm, hist_v)
    lane = lax.iota(jnp.int32, 16)
    lane8 = jnp.bitwise_and(lane, NHL - 1)
    mlo = lane < NHL
    mhi = lane >= NHL
    ones = jnp.full((16,), 1.0, jnp.float32)

    def group(j, carry):
        pltpu.sync_copy(dstp_hbm.at[wid, pl.ds(j * DEG_B, DEG_B)], idx_v)

        def chunk(q, carry2):
            for m in range(K // 16):
                v = idx_v[q, pl.ds(16 * m, 16)] * NHL + lane8
                plsc.addupdate_scatter(hist_v, [v], ones, mask=mlo)
                plsc.addupdate_scatter(hist_v, [v], ones, mask=mhi)
            return carry2

        lax.fori_loop(0, DEG_B, chunk, 0)
        return carry

    lax.fori_loop(0, NCHUNK // DEG_B, group, 0)
    pltpu.sync_copy(hist_v, degp_hbm.at[wid])


_deg_kernel = pl.kernel(_deg_body, **_DEG_KERNEL_PARAMS)


_BM = 1280


def _linear_body(x_ref, w_ref, degt_ref, g_ref):
    deg = jnp.sum(degt_ref[...], axis=1, keepdims=True) + 1.0
    dis = lax.rsqrt(deg)
    h = jnp.dot(x_ref[...], w_ref[...], preferred_element_type=jnp.float32)
    g_ref[...] = h * dis


def _linear(x_pad, W, degt):
    return pl.pallas_call(
        _linear_body,
        grid=(NPAD // _BM,),
        in_specs=[
            pl.BlockSpec((_BM, D), lambda i: (i, 0)),
            pl.BlockSpec((D, D), lambda i: (0, 0)),
            pl.BlockSpec((_BM, NW * NHL), lambda i: (i, 0)),
        ],
        out_specs=pl.BlockSpec((_BM, D), lambda i: (i, 0)),
        out_shape=jax.ShapeDtypeStruct((NPAD, D), jnp.float32),
    )(x_pad, W, degt)


_AGG_KERNEL_PARAMS = dict(
    out_type=jax.ShapeDtypeStruct((NC, NPAD, D), jnp.float32),
    mesh=_mesh,
    scratch_types=[
        pltpu.VMEM((2, K), jnp.int32),
        pltpu.VMEM((2, K), jnp.int32),
        pltpu.VMEM((K,), jnp.int32),
        pltpu.VMEM((K, D), jnp.float32),
        pltpu.VMEM((K, D), jnp.float32),
        pltpu.VMEM_SHARED((NPAD, D), jnp.float32),
        pltpu.SemaphoreType.DMA,
        pltpu.SemaphoreType.DMA,
        pltpu.SemaphoreType.DMA,
        pltpu.SemaphoreType.DMA,
    ],
)


def _agg_body(g_hbm, srcp_hbm, dstp_hbm, zacc_hbm, pad_hbm, accp_hbm,
              sidx2, didx2, padv, r0, r1, acc_sh, gs0, gs1, ss0, ss1):
    c = lax.axis_index("c")
    s = lax.axis_index("s")
    wid = c * NS + s
    pltpu.sync_copy(zacc_hbm, acc_sh.at[pl.ds(s * ROWS_PER_TILE, ROWS_PER_TILE)])
    plsc.subcore_barrier()

    pltpu.sync_copy(pad_hbm, padv)
    pltpu.async_copy(r0, acc_sh.at[padv], ss0, add=True)
    pltpu.async_copy(r1, acc_sh.at[padv], ss1, add=True)

    def pair(t, carry):
        pltpu.sync_copy(srcp_hbm.at[wid, pl.ds(2 * t, 2)], sidx2)
        pltpu.make_async_copy(r0, acc_sh.at[padv], ss0).wait()
        cp_a = pltpu.async_copy(g_hbm.at[sidx2.at[0]], r0, gs0)
        pltpu.make_async_copy(r1, acc_sh.at[padv], ss1).wait()
        cp_b = pltpu.async_copy(g_hbm.at[sidx2.at[1]], r1, gs1)
        pltpu.sync_copy(dstp_hbm.at[wid, pl.ds(2 * t, 2)], didx2)
        cp_a.wait()
        pltpu.async_copy(r0, acc_sh.at[didx2.at[0]], ss0, add=True)
        cp_b.wait()
        pltpu.async_copy(r1, acc_sh.at[didx2.at[1]], ss1, add=True)
        return carry

    lax.fori_loop(0, NCHUNK // 2, pair, 0)
    pltpu.make_async_copy(r0, acc_sh.at[padv], ss0).wait()
    pltpu.make_async_copy(r1, acc_sh.at[padv], ss1).wait()
    plsc.subcore_barrier()
    off = s * ROWS_PER_TILE
    pltpu.sync_copy(acc_sh.at[pl.ds(off, ROWS_PER_TILE)],
                    accp_hbm.at[c, pl.ds(off, ROWS_PER_TILE)])


_agg_kernel = pl.kernel(_agg_body, **_AGG_KERNEL_PARAMS)


def _combine_body(accp_ref, g_ref, degt_ref, b_ref, out_ref):
    deg = jnp.sum(degt_ref[...], axis=1, keepdims=True) + 1.0
    dis = lax.rsqrt(deg)
    total = accp_ref[0] + accp_ref[1] + g_ref[...]
    out_ref[...] = total * dis + b_ref[...]


def _combine(accp, g, degt, b2d):
    return pl.pallas_call(
        _combine_body,
        grid=(NPAD // _BM,),
        in_specs=[
            pl.BlockSpec((NC, _BM, D), lambda i: (0, i, 0)),
            pl.BlockSpec((_BM, D), lambda i: (i, 0)),
            pl.BlockSpec((_BM, NW * NHL), lambda i: (i, 0)),
            pl.BlockSpec((1, D), lambda i: (0, 0)),
        ],
        out_specs=pl.BlockSpec((_BM, D), lambda i: (i, 0)),
        out_shape=jax.ShapeDtypeStruct((NPAD, D), jnp.float32),
    )(accp, g, degt, b2d)


def kernel(x, edge_index, W, b):
    src = edge_index[0].astype(jnp.int32)
    dst = edge_index[1].astype(jnp.int32)
    pad_slots = NCHUNK * K - EPT
    srcp = jnp.pad(src.reshape(NW, EPT), ((0, 0), (0, pad_slots)),
                   constant_values=PAD_ROW).reshape(NW, NCHUNK, K)
    dstp = jnp.pad(dst.reshape(NW, EPT), ((0, 0), (0, pad_slots)),
                   constant_values=PAD_ROW).reshape(NW, NCHUNK, K)
    x_pad = jnp.pad(x, ((0, NPAD - N_NODES), (0, 0)))

    zhist = jnp.zeros((NHL * NPAD,), jnp.float32)
    zacc = jnp.zeros((ROWS_PER_TILE, D), jnp.float32)

    pad_idx = jnp.full((K,), PAD_ROW, jnp.int32)

    degp = _deg_kernel(dstp, zhist)
    degt = degp.reshape(NW, NPAD, NHL).transpose(1, 0, 2).reshape(NPAD, NW * NHL)
    g = _linear(x_pad, W, degt)
    accp = _agg_kernel(g, srcp, dstp, zacc, pad_idx)
    out = _combine(accp, g, degt, b.reshape(1, D))
    return out[:N_NODES]


# speedup vs baseline: 1.0178x; 1.0178x over previous
import functools

import jax
import jax.numpy as jnp
from jax import lax
from jax.experimental import pallas as pl
from jax.experimental.pallas import tpu as pltpu
from jax.experimental.pallas import tpu_sc as plsc

N_NODES = 10000
N_EDGES = 320000
D = 128

NC, NS = 2, 16
NW = NC * NS
NPAD = 10240
ROWS_PER_TILE = NPAD // NS
K = 128
EPT = N_EDGES // NW
NCHUNK = 80
KA = 64
NCHA = 160
PAD_ROW = NPAD - 1

_mesh = plsc.VectorSubcoreMesh(core_axis_name="c", subcore_axis_name="s")


NHL = 8

DEG_B = 4

_DEG_KERNEL_PARAMS = dict(
    out_type=jax.ShapeDtypeStruct((NW, NHL * NPAD), jnp.float32),
    mesh=_mesh,
    compiler_params=pltpu.CompilerParams(needs_layout_passes=False),
    scratch_types=[
        pltpu.VMEM((DEG_B, K), jnp.int32),
        pltpu.VMEM((NHL * NPAD,), jnp.float32),
    ],
)


def _deg_body(dstp_hbm, zhist_hbm, degp_hbm, idx_v, hist_v):
    c = lax.axis_index("c")
    s = lax.axis_index("s")
    wid = c * NS + s
    pltpu.sync_copy(zhist_hbm, hist_v)
    lane = lax.iota(jnp.int32, 16)
    lane8 = jnp.bitwise_and(lane, NHL - 1)
    mlo = lane < NHL
    mhi = lane >= NHL
    ones = jnp.full((16,), 1.0, jnp.float32)

    def group(j, carry):
        pltpu.sync_copy(dstp_hbm.at[wid, pl.ds(j * DEG_B, DEG_B)], idx_v)

        def chunk(q, carry2):
            for m in range(K // 16):
                v = idx_v[q, pl.ds(16 * m, 16)] * NHL + lane8
                plsc.addupdate_scatter(hist_v, [v], ones, mask=mlo)
                plsc.addupdate_scatter(hist_v, [v], ones, mask=mhi)
            return carry2

        lax.fori_loop(0, DEG_B, chunk, 0)
        return carry

    lax.fori_loop(0, NCHUNK // DEG_B, group, 0)
    pltpu.sync_copy(hist_v, degp_hbm.at[wid])


_deg_kernel = pl.kernel(_deg_body, **_DEG_KERNEL_PARAMS)


_BM = 1280


def _linear_body(x_ref, w_ref, degt_ref, g_ref):
    deg = jnp.sum(degt_ref[...], axis=1, keepdims=True) + 1.0
    dis = lax.rsqrt(deg)
    h = jnp.dot(x_ref[...], w_ref[...], preferred_element_type=jnp.float32)
    g_ref[...] = h * dis


def _linear(x_pad, W, degt):
    return pl.pallas_call(
        _linear_body,
        grid=(NPAD // _BM,),
        in_specs=[
            pl.BlockSpec((_BM, D), lambda i: (i, 0)),
            pl.BlockSpec((D, D), lambda i: (0, 0)),
            pl.BlockSpec((_BM, NW * NHL), lambda i: (i, 0)),
        ],
        out_specs=pl.BlockSpec((_BM, D), lambda i: (i, 0)),
        out_shape=jax.ShapeDtypeStruct((NPAD, D), jnp.float32),
    )(x_pad, W, degt)


_AGG_KERNEL_PARAMS = dict(
    out_type=jax.ShapeDtypeStruct((NC, NPAD, D), jnp.float32),
    mesh=_mesh,
    scratch_types=[
        pltpu.VMEM((4, KA), jnp.int32),
        pltpu.VMEM((4, KA), jnp.int32),
        pltpu.VMEM((KA,), jnp.int32),
        pltpu.VMEM((KA, D), jnp.float32),
        pltpu.VMEM((KA, D), jnp.float32),
        pltpu.VMEM((KA, D), jnp.float32),
        pltpu.VMEM((KA, D), jnp.float32),
        pltpu.VMEM_SHARED((NPAD, D), jnp.float32),
        pltpu.SemaphoreType.DMA,
        pltpu.SemaphoreType.DMA,
        pltpu.SemaphoreType.DMA,
        pltpu.SemaphoreType.DMA,
        pltpu.SemaphoreType.DMA,
        pltpu.SemaphoreType.DMA,
        pltpu.SemaphoreType.DMA,
        pltpu.SemaphoreType.DMA,
    ],
)


def _agg_body(g_hbm, srcp_hbm, dstp_hbm, zacc_hbm, pad_hbm, accp_hbm,
              sidx, didx, padv, r0, r1, r2, r3, acc_sh,
              gs0, gs1, gs2, gs3, ss0, ss1, ss2, ss3):
    c = lax.axis_index("c")
    s = lax.axis_index("s")
    wid = c * NS + s
    rows = [r0, r1, r2, r3]
    gsem = [gs0, gs1, gs2, gs3]
    ssem = [ss0, ss1, ss2, ss3]
    pltpu.sync_copy(zacc_hbm, acc_sh.at[pl.ds(s * ROWS_PER_TILE, ROWS_PER_TILE)])
    plsc.subcore_barrier()

    pltpu.sync_copy(pad_hbm, padv)
    for b in range(4):
        pltpu.async_copy(rows[b], acc_sh.at[padv], ssem[b], add=True)

    def body(t, carry):
        j0 = 4 * t
        pltpu.sync_copy(srcp_hbm.at[wid, pl.ds(j0, 4)], sidx)
        gd = []
        for b in range(4):
            pltpu.make_async_copy(rows[b], acc_sh.at[padv], ssem[b]).wait()
            gd.append(pltpu.async_copy(g_hbm.at[sidx.at[b]], rows[b],
                                       gsem[b]))
        pltpu.sync_copy(dstp_hbm.at[wid, pl.ds(j0, 4)], didx)
        for b in range(4):
            gd[b].wait()
            pltpu.async_copy(rows[b], acc_sh.at[didx.at[b]], ssem[b],
                             add=True)
        return carry

    lax.fori_loop(0, NCHA // 4, body, 0)
    for b in range(4):
        pltpu.make_async_copy(rows[b], acc_sh.at[padv], ssem[b]).wait()
    plsc.subcore_barrier()
    off = s * ROWS_PER_TILE
    pltpu.sync_copy(acc_sh.at[pl.ds(off, ROWS_PER_TILE)],
                    accp_hbm.at[c, pl.ds(off, ROWS_PER_TILE)])


_agg_kernel = pl.kernel(_agg_body, **_AGG_KERNEL_PARAMS)


def _combine_body(accp_ref, g_ref, degt_ref, b_ref, out_ref):
    deg = jnp.sum(degt_ref[...], axis=1, keepdims=True) + 1.0
    dis = lax.rsqrt(deg)
    total = accp_ref[0] + accp_ref[1] + g_ref[...]
    out_ref[...] = total * dis + b_ref[...]


def _combine(accp, g, degt, b2d):
    return pl.pallas_call(
        _combine_body,
        grid=(NPAD // _BM,),
        in_specs=[
            pl.BlockSpec((NC, _BM, D), lambda i: (0, i, 0)),
            pl.BlockSpec((_BM, D), lambda i: (i, 0)),
            pl.BlockSpec((_BM, NW * NHL), lambda i: (i, 0)),
            pl.BlockSpec((1, D), lambda i: (0, 0)),
        ],
        out_specs=pl.BlockSpec((_BM, D), lambda i: (i, 0)),
        out_shape=jax.ShapeDtypeStruct((NPAD, D), jnp.float32),
    )(accp, g, degt, b2d)


def kernel(x, edge_index, W, b):
    src = edge_index[0].astype(jnp.int32)
    dst = edge_index[1].astype(jnp.int32)
    pad_slots = NCHUNK * K - EPT
    srcf = jnp.pad(src.reshape(NW, EPT), ((0, 0), (0, pad_slots)),
                   constant_values=PAD_ROW)
    dstf = jnp.pad(dst.reshape(NW, EPT), ((0, 0), (0, pad_slots)),
                   constant_values=PAD_ROW)
    x_pad = jnp.pad(x, ((0, NPAD - N_NODES), (0, 0)))

    zhist = jnp.zeros((NHL * NPAD,), jnp.float32)
    zacc = jnp.zeros((ROWS_PER_TILE, D), jnp.float32)

    pad_idx = jnp.full((KA,), PAD_ROW, jnp.int32)

    degp = _deg_kernel(dstf.reshape(NW, NCHUNK, K), zhist)
    degt = degp.reshape(NW, NPAD, NHL).transpose(1, 0, 2).reshape(NPAD, NW * NHL)
    g = _linear(x_pad, W, degt)
    accp = _agg_kernel(g, srcf.reshape(NW, NCHA, KA),
                       dstf.reshape(NW, NCHA, KA), zacc, pad_idx)
    out = _combine(accp, g, degt, b.reshape(1, D))
    return out[:N_NODES]
